# Initial kernel scaffold; baseline (speedup 1.0000x reference)
import functools

import jax
import jax.numpy as jnp
from jax import lax
from jax.experimental import pallas as pl
from jax.experimental.pallas import tpu as pltpu
from jax.experimental.pallas import tpu_sc as plsc

B = 16384
NUM_LABELS = 500
NUM_MODALITIES = 4
D = 768

_INFO = plsc.get_sparse_core_info()
NC = _INFO.num_cores
NS = _INFO.num_subcores
L = _INFO.num_lanes
NW = NC * NS
BPW = B // NW
CHUNK = 64
NCHUNK = BPW // CHUNK


def _combine_body(mod_ref, label_ref, out_ref):
    out_ref[...] = label_ref[...] + mod_ref[...]


def _build_combined(modality_embed, label_embed):
    return pl.pallas_call(
        _combine_body,
        grid=(NUM_MODALITIES,),
        in_specs=[
            pl.BlockSpec((1, D), lambda j: (j, 0)),
            pl.BlockSpec((NUM_LABELS, D), lambda j: (0, 0)),
        ],
        out_specs=pl.BlockSpec((NUM_LABELS, D), lambda j: (j, 0)),
        out_shape=jax.ShapeDtypeStruct((NUM_MODALITIES * NUM_LABELS, D), jnp.float32),
    )(modality_embed, label_embed)


_MESH = plsc.VectorSubcoreMesh(core_axis_name="c", subcore_axis_name="s")


@functools.partial(
    pl.kernel,
    mesh=_MESH,
    out_type=jax.ShapeDtypeStruct((B, D), jnp.float32),
    scratch_types=[
        pltpu.VMEM((BPW,), jnp.int32),
        pltpu.VMEM((BPW,), jnp.int32),
        pltpu.VMEM((BPW,), jnp.int32),
        pltpu.VMEM((2, CHUNK, D), jnp.float32),
        pltpu.SemaphoreType.DMA,
        pltpu.SemaphoreType.DMA,
    ],
)
def _sc_gather(comb_hbm, lc_hbm, mc_hbm, out_hbm,
               lc_v, mc_v, idx_v, rows_v, sem0, sem1):
    wid = lax.axis_index("s") * NC + lax.axis_index("c")
    base = wid * BPW
    pltpu.sync_copy(lc_hbm.at[pl.ds(base, BPW)], lc_v)
    pltpu.sync_copy(mc_hbm.at[pl.ds(base, BPW)], mc_v)
    for j in range(BPW // L):
        sl = pl.ds(j * L, L)
        idx_v[sl] = mc_v[sl] * NUM_LABELS + lc_v[sl]
    sems = (sem0, sem1)

    def _start(c):
        return pltpu.async_copy(
            comb_hbm.at[idx_v.at[pl.ds(c * CHUNK, CHUNK)]],
            rows_v.at[c % 2], sems[c % 2])

    pending = _start(0)
    for c in range(NCHUNK):
        nxt = _start(c + 1) if c + 1 < NCHUNK else None
        pending.wait()
        pltpu.sync_copy(rows_v.at[c % 2],
                        out_hbm.at[pl.ds(base + c * CHUNK, CHUNK)])
        pending = nxt


def kernel(label_code, modality_code, label_embed, modality_embed):
    lc = label_code.astype(jnp.int32)
    mc = modality_code.astype(jnp.int32)
    combined = _build_combined(modality_embed, label_embed)
    return _sc_gather(combined, lc, mc)


# speedup vs baseline: 2.9046x; 2.9046x over previous
import functools

import jax
import jax.numpy as jnp
from jax import lax
from jax.experimental import pallas as pl
from jax.experimental.pallas import tpu as pltpu
from jax.experimental.pallas import tpu_sc as plsc

B = 16384
NUM_LABELS = 500
NUM_MODALITIES = 4
D = 768

_INFO = plsc.get_sparse_core_info()
NC = _INFO.num_cores
NS = _INFO.num_subcores
L = _INFO.num_lanes
NW = NC * NS
BPW = B // NW
CHUNK = 64
NCHUNK = BPW // CHUNK


def _combine_body(mod_ref, label_ref, out_ref):
    j = pl.program_id(0)
    out_ref[...] = (label_ref[...] + mod_ref[pl.ds(j, 1), :])[None]


def _build_combined(modality_embed, label_embed):
    return pl.pallas_call(
        _combine_body,
        grid=(NUM_MODALITIES,),
        in_specs=[
            pl.BlockSpec((NUM_MODALITIES, D), lambda j: (0, 0)),
            pl.BlockSpec((NUM_LABELS, D), lambda j: (0, 0)),
        ],
        out_specs=pl.BlockSpec((1, NUM_LABELS, D), lambda j: (j, 0, 0)),
        out_shape=jax.ShapeDtypeStruct(
            (NUM_MODALITIES, NUM_LABELS, D), jnp.float32),
    )(modality_embed, label_embed)


_MESH = plsc.VectorSubcoreMesh(core_axis_name="c", subcore_axis_name="s")


@functools.partial(
    pl.kernel,
    mesh=_MESH,
    out_type=jax.ShapeDtypeStruct((B, D), jnp.float32),
    scratch_types=[
        pltpu.VMEM((BPW,), jnp.int32),
        pltpu.VMEM((BPW,), jnp.int32),
        pltpu.VMEM((BPW,), jnp.int32),
        pltpu.VMEM((2, CHUNK, D), jnp.float32),
        pltpu.SemaphoreType.DMA,
        pltpu.SemaphoreType.DMA,
    ],
)
def _sc_gather(comb_hbm, lc_hbm, mc_hbm, out_hbm,
               lc_v, mc_v, idx_v, rows_v, sem0, sem1):
    wid = lax.axis_index("s") * NC + lax.axis_index("c")
    base = wid * BPW
    pltpu.sync_copy(lc_hbm.at[pl.ds(base, BPW)], lc_v)
    pltpu.sync_copy(mc_hbm.at[pl.ds(base, BPW)], mc_v)
    for j in range(BPW // L):
        sl = pl.ds(j * L, L)
        idx_v[sl] = mc_v[sl] * NUM_LABELS + lc_v[sl]
    sems = (sem0, sem1)

    def _start(c):
        return pltpu.async_copy(
            comb_hbm.at[idx_v.at[pl.ds(c * CHUNK, CHUNK)]],
            rows_v.at[c % 2], sems[c % 2])

    pending = _start(0)
    for c in range(NCHUNK):
        nxt = _start(c + 1) if c + 1 < NCHUNK else None
        pending.wait()
        pltpu.sync_copy(rows_v.at[c % 2],
                        out_hbm.at[pl.ds(base + c * CHUNK, CHUNK)])
        pending = nxt


def kernel(label_code, modality_code, label_embed, modality_embed):
    lc = label_code.astype(jnp.int32)
    mc = modality_code.astype(jnp.int32)
    combined = _build_combined(modality_embed, label_embed)
    combined = combined.reshape(NUM_MODALITIES * NUM_LABELS, D)
    return _sc_gather(combined, lc, mc)


# speedup vs baseline: 2.9274x; 1.0079x over previous
import functools

import jax
import jax.numpy as jnp
from jax import lax
from jax.experimental import pallas as pl
from jax.experimental.pallas import tpu as pltpu
from jax.experimental.pallas import tpu_sc as plsc

B = 16384
NUM_LABELS = 500
NUM_MODALITIES = 4
D = 768

_INFO = plsc.get_sparse_core_info()
NC = _INFO.num_cores
NS = _INFO.num_subcores
L = _INFO.num_lanes
NW = NC * NS
BPW = B // NW
CHUNK = 32
NCHUNK = BPW // CHUNK
NBUF = 4
LAG = 2


def _combine_body(mod_ref, label_ref, out_ref):
    j = pl.program_id(0)
    out_ref[...] = (label_ref[...] + mod_ref[pl.ds(j, 1), :])[None]


def _build_combined(modality_embed, label_embed):
    return pl.pallas_call(
        _combine_body,
        grid=(NUM_MODALITIES,),
        in_specs=[
            pl.BlockSpec((NUM_MODALITIES, D), lambda j: (0, 0)),
            pl.BlockSpec((NUM_LABELS, D), lambda j: (0, 0)),
        ],
        out_specs=pl.BlockSpec((1, NUM_LABELS, D), lambda j: (j, 0, 0)),
        out_shape=jax.ShapeDtypeStruct(
            (NUM_MODALITIES, NUM_LABELS, D), jnp.float32),
    )(modality_embed, label_embed)


_MESH = plsc.VectorSubcoreMesh(core_axis_name="c", subcore_axis_name="s")


@functools.partial(
    pl.kernel,
    mesh=_MESH,
    out_type=jax.ShapeDtypeStruct((B, D), jnp.float32),
    scratch_types=[
        pltpu.VMEM((BPW,), jnp.int32),
        pltpu.VMEM((BPW,), jnp.int32),
        pltpu.VMEM((BPW,), jnp.int32),
        pltpu.VMEM((NBUF, CHUNK, D), jnp.float32),
    ] + [pltpu.SemaphoreType.DMA] * (2 * NBUF),
)
def _sc_gather(comb_hbm, lc_hbm, mc_hbm, out_hbm,
               lc_v, mc_v, idx_v, rows_v, *sems):
    gsems, wsems = sems[:NBUF], sems[NBUF:]
    wid = lax.axis_index("s") * NC + lax.axis_index("c")
    base = wid * BPW
    pltpu.sync_copy(lc_hbm.at[pl.ds(base, BPW)], lc_v)
    pltpu.sync_copy(mc_hbm.at[pl.ds(base, BPW)], mc_v)
    for j in range(BPW // L):
        sl = pl.ds(j * L, L)
        idx_v[sl] = mc_v[sl] * NUM_LABELS + lc_v[sl]

    gh = [None] * NBUF
    wh = [None] * NBUF
    for t in range(NCHUNK + LAG):
        if t < NCHUNK:
            b = t % NBUF
            if wh[b] is not None:
                wh[b].wait()
                wh[b] = None
            gh[b] = pltpu.async_copy(
                comb_hbm.at[idx_v.at[pl.ds(t * CHUNK, CHUNK)]],
                rows_v.at[b], gsems[b])
        d = t - LAG
        if d >= 0:
            b = d % NBUF
            gh[b].wait()
            wh[b] = pltpu.async_copy(
                rows_v.at[b],
                out_hbm.at[pl.ds(base + d * CHUNK, CHUNK)], wsems[b])
    for b in range(NBUF):
        if wh[b] is not None:
            wh[b].wait()


def kernel(label_code, modality_code, label_embed, modality_embed):
    lc = label_code.astype(jnp.int32)
    mc = modality_code.astype(jnp.int32)
    combined = _build_combined(modality_embed, label_embed)
    combined = combined.reshape(NUM_MODALITIES * NUM_LABELS, D)
    return _sc_gather(combined, lc, mc)


# speedup vs baseline: 3.1747x; 1.0845x over previous
import functools

import jax
import jax.numpy as jnp
from jax import lax
from jax.experimental import pallas as pl
from jax.experimental.pallas import tpu as pltpu
from jax.experimental.pallas import tpu_sc as plsc

B = 16384
NUM_LABELS = 500
NUM_MODALITIES = 4
D = 768

_INFO = plsc.get_sparse_core_info()
NC = _INFO.num_cores
NS = _INFO.num_subcores
L = _INFO.num_lanes
NW = NC * NS
BPW = B // NW
CHUNK = 32
NCHUNK = BPW // CHUNK
NBUF = 4
LAG = 2


LPAD = 512


def _combine_body(mod_ref, label_ref, out_ref):
    for m in range(NUM_MODALITIES):
        out_ref[pl.ds(m * LPAD, NUM_LABELS), :] = (
            label_ref[...] + mod_ref[pl.ds(m, 1), :])


def _build_combined(modality_embed, label_embed):
    return pl.pallas_call(
        _combine_body,
        out_shape=jax.ShapeDtypeStruct(
            (NUM_MODALITIES * LPAD, D), jnp.float32),
    )(modality_embed, label_embed)


_MESH = plsc.VectorSubcoreMesh(core_axis_name="c", subcore_axis_name="s")


@functools.partial(
    pl.kernel,
    mesh=_MESH,
    out_type=jax.ShapeDtypeStruct((B, D), jnp.float32),
    scratch_types=[
        pltpu.VMEM((BPW,), jnp.int32),
        pltpu.VMEM((BPW,), jnp.int32),
        pltpu.VMEM((BPW,), jnp.int32),
        pltpu.VMEM((NBUF, CHUNK, D), jnp.float32),
    ] + [pltpu.SemaphoreType.DMA] * (2 * NBUF),
)
def _sc_gather(comb_hbm, lc_hbm, mc_hbm, out_hbm,
               lc_v, mc_v, idx_v, rows_v, *sems):
    gsems, wsems = sems[:NBUF], sems[NBUF:]
    wid = lax.axis_index("s") * NC + lax.axis_index("c")
    base = wid * BPW
    pltpu.sync_copy(lc_hbm.at[pl.ds(base, BPW)], lc_v)
    pltpu.sync_copy(mc_hbm.at[pl.ds(base, BPW)], mc_v)
    for j in range(BPW // L):
        sl = pl.ds(j * L, L)
        idx_v[sl] = mc_v[sl] * LPAD + lc_v[sl]

    gh = [None] * NBUF
    wh = [None] * NBUF
    for t in range(NCHUNK + LAG):
        if t < NCHUNK:
            b = t % NBUF
            if wh[b] is not None:
                wh[b].wait()
                wh[b] = None
            gh[b] = pltpu.async_copy(
                comb_hbm.at[idx_v.at[pl.ds(t * CHUNK, CHUNK)]],
                rows_v.at[b], gsems[b])
        d = t - LAG
        if d >= 0:
            b = d % NBUF
            gh[b].wait()
            wh[b] = pltpu.async_copy(
                rows_v.at[b],
                out_hbm.at[pl.ds(base + d * CHUNK, CHUNK)], wsems[b])
    for b in range(NBUF):
        if wh[b] is not None:
            wh[b].wait()


def kernel(label_code, modality_code, label_embed, modality_embed):
    lc = label_code.astype(jnp.int32)
    mc = modality_code.astype(jnp.int32)
    combined = _build_combined(modality_embed, label_embed)
    return _sc_gather(combined, lc, mc)
